# baseline (speedup 1.0000x reference)
import functools

import jax
import jax.numpy as jnp
from jax.experimental import pallas as pl
from jax.experimental.pallas import tpu as pltpu

N_I = 256
N_P = 8192
N_C = 9
I_BLK = 32
N_STEPS = N_I // I_BLK


def _rot_plane_kernel(w_ref, x_ref, x_hbm, o_hbm, obuf, out_sem, cp_sem):
    i = pl.program_id(0)
    slot = jax.lax.rem(i, 2)

    def plane_copy(k):
        return pltpu.make_async_copy(
            x_hbm.at[3 + k], o_hbm.at[3 + k], cp_sem.at[k]
        )

    @pl.when(i == 0)
    def _():
        for k in range(N_C - 3):
            plane_copy(k).start()

    def out_copy(step, s):
        return pltpu.make_async_copy(
            obuf.at[s],
            o_hbm.at[pl.ds(0, 3), pl.ds(step * I_BLK, I_BLK)],
            out_sem.at[s],
        )

    @pl.when(i >= 2)
    def _():
        out_copy(i - 2, slot).wait()

    w = w_ref[...]
    for d in range(3):
        acc = x_ref[0] * w[:, 3 * d : 3 * d + 1]
        acc += x_ref[1] * w[:, 3 * d + 1 : 3 * d + 2]
        acc += x_ref[2] * w[:, 3 * d + 2 : 3 * d + 3]
        obuf[slot, d] = acc
    out_copy(i, slot).start()

    @pl.when(i == N_STEPS - 1)
    def _():
        out_copy(i - 1, 1 - slot).wait()
        out_copy(i, slot).wait()
        for k in range(N_C - 3):
            plane_copy(k).wait()


@functools.partial(jax.jit, static_argnames=("interpret",))
def kernel(points_colored_instance, rot_mats, interpret=False):
    xt = jnp.transpose(points_colored_instance, (2, 0, 1))
    w = rot_mats.reshape(N_I, 9)
    out = pl.pallas_call(
        _rot_plane_kernel,
        grid=(N_STEPS,),
        in_specs=[
            pl.BlockSpec((I_BLK, 9), lambda i: (i, 0)),
            pl.BlockSpec((3, I_BLK, N_P), lambda i: (0, i, 0)),
            pl.BlockSpec(memory_space=pl.ANY),
        ],
        out_specs=pl.BlockSpec(memory_space=pl.ANY),
        out_shape=jax.ShapeDtypeStruct((N_C, N_I, N_P), jnp.float32),
        scratch_shapes=[
            pltpu.VMEM((2, 3, I_BLK, N_P), jnp.float32),
            pltpu.SemaphoreType.DMA((2,)),
            pltpu.SemaphoreType.DMA((N_C - 3,)),
        ],
        interpret=interpret,
    )(w, xt, xt)
    return jnp.transpose(out, (1, 2, 0))


# speedup vs baseline: 14.8350x; 14.8350x over previous
import functools

import jax
import jax.numpy as jnp
from jax import lax
from jax.experimental import pallas as pl
from jax.experimental.pallas import tpu as pltpu
from jax.experimental.pallas import tpu_sc as plsc

N_I = 256
N_P = 8192
N_C = 9
NW = 32
I_W = N_I // NW
CHUNK = 1024
NQ = N_P // CHUNK
CP = 4096
NH = N_P // CP
LANES = 16


def _sc_body(w_hbm, x_hbm, o_hbm, wvb, xb, ob, cpb, lsem, ssem, clsem, cssem):
    wid = lax.axis_index("s") * 2 + lax.axis_index("c")
    base = wid * I_W

    pltpu.sync_copy(w_hbm.at[pl.ds(base, I_W)], wvb)

    def rot_load(q, c):
        return pltpu.make_async_copy(
            x_hbm.at[c, pl.ds(base, I_W), pl.ds(q * CHUNK, CHUNK)],
            xb.at[c], lsem)

    def rot_store(q, d):
        return pltpu.make_async_copy(
            ob.at[d],
            o_hbm.at[d, pl.ds(base, I_W), pl.ds(q * CHUNK, CHUNK)], ssem)

    def rot_q(q, carry):
        for c in range(3):
            rot_load(q, c).start()
        @pl.when(q > 0)
        def _():
            for d in range(3):
                rot_store(q - 1, d).wait()
        for c in range(3):
            rot_load(q, c).wait()
        for ii in range(I_W):
            wv = [wvb[ii, k] for k in range(9)]

            def grp(g, carry2):
                idx = pl.ds(g * LANES, LANES)
                x0 = xb[0, ii, idx]
                x1 = xb[1, ii, idx]
                x2 = xb[2, ii, idx]
                for d in range(3):
                    ob[d, ii, idx] = (x0 * wv[3 * d] + x1 * wv[3 * d + 1]
                                      + x2 * wv[3 * d + 2])
                return carry2

            lax.fori_loop(0, CHUNK // LANES, grp, 0, unroll=4)
        for d in range(3):
            rot_store(q, d).start()
        return carry

    lax.fori_loop(0, NQ, rot_q, 0)

    def cp_load(t, slot):
        plane = 3 + t // NH
        half = lax.rem(t, NH)
        return pltpu.make_async_copy(
            x_hbm.at[plane, pl.ds(base, I_W), pl.ds(half * CP, CP)],
            cpb.at[slot], clsem.at[slot])

    def cp_store(t, slot):
        plane = 3 + t // NH
        half = lax.rem(t, NH)
        return pltpu.make_async_copy(
            cpb.at[slot],
            o_hbm.at[plane, pl.ds(base, I_W), pl.ds(half * CP, CP)],
            cssem.at[slot])

    def cp_t(t, carry):
        slot = lax.rem(t, 2)
        @pl.when(t >= 2)
        def _():
            cp_store(t - 2, slot).wait()
        cp_load(t, slot).start()
        cp_load(t, slot).wait()
        cp_store(t, slot).start()
        return carry

    n_t = (N_C - 3) * NH
    lax.fori_loop(0, n_t, cp_t, 0)
    for d in range(3):
        rot_store(NQ - 1, d).wait()
    cp_store(n_t - 2, lax.rem(n_t - 2, 2)).wait()
    cp_store(n_t - 1, lax.rem(n_t - 1, 2)).wait()


@functools.partial(jax.jit, static_argnames=("interpret",))
def kernel(points_colored_instance, rot_mats, interpret=False):
    xt = jnp.transpose(points_colored_instance, (2, 0, 1))
    w = rot_mats.reshape(N_I, 9)
    wsplat = jnp.broadcast_to(w[:, :, None], (N_I, 9, LANES))
    mesh = plsc.VectorSubcoreMesh(core_axis_name="c", subcore_axis_name="s")
    run = pl.kernel(
        _sc_body,
        out_type=jax.ShapeDtypeStruct((N_C, N_I, N_P), jnp.float32),
        mesh=mesh,
        scratch_types=[
            pltpu.VMEM((I_W, 9, LANES), jnp.float32),
            pltpu.VMEM((3, I_W, CHUNK), jnp.float32),
            pltpu.VMEM((3, I_W, CHUNK), jnp.float32),
            pltpu.VMEM((2, I_W, CP), jnp.float32),
            pltpu.SemaphoreType.DMA,
            pltpu.SemaphoreType.DMA,
            pltpu.SemaphoreType.DMA((2,)),
            pltpu.SemaphoreType.DMA((2,)),
        ],
        interpret=interpret,
    )
    out = run(wsplat, xt)
    return jnp.transpose(out, (1, 2, 0))
